# Initial kernel scaffold; baseline (speedup 1.0000x reference)
import jax
import jax.numpy as jnp
from jax.experimental import pallas as pl


def kernel(indices, weight):
    raise NotImplementedError("write your pallas kernel here")


# speedup vs baseline: 25.4321x; 25.4321x over previous
import functools

import numpy as np
import jax
import jax.numpy as jnp
from jax import lax
from jax.experimental import pallas as pl
from jax.experimental.pallas import tpu as pltpu
from jax.experimental.pallas import tpu_sc as plsc

_DIM = 64
_WSIZE = 1_048_576
_BATCH = 16_384
_M = _WSIZE - 1

_NW = 32
_BPW = _BATCH // _NW
_NCH = _BPW // 16
_ROWS = _BPW * _DIM // 128


def _hash_constants():
    r = np.random.RandomState(1024)
    rn = np.concatenate(
        [np.array([2038074743]), r.randint(0, 2038074743, (10,))]
    ).astype(np.int64)
    return int(rn[0]), int(rn[1]), int(rn[2]), int(rn[3])


_P, _A, _B, _C = _hash_constants()
_KD = [int((((d + 1) * _B + _C) % _P) - 2**31) for d in range(_DIM)]
_T0 = np.array([(k * _A) % _P for k in range(1024)], dtype=np.int64).astype(np.int32)
_T1B = np.array(
    [((k * 1024 * _A) % _P) - 2**31 for k in range(1024)], dtype=np.int64
).astype(np.int32)
_PBIAS = np.int32(_P - 2**31)
_R31 = np.int32(2**31 - _P)
_MIN32 = np.int32(-(2**31))

_mesh = plsc.VectorSubcoreMesh(core_axis_name="c", subcore_axis_name="s")


@functools.partial(
    pl.kernel,
    out_type=jax.ShapeDtypeStruct((_BATCH * _DIM // 128, 128), jnp.float32),
    mesh=_mesh,
    compiler_params=pltpu.CompilerParams(needs_layout_passes=False),
    scratch_types=[
        pltpu.VMEM((_BPW,), jnp.int32),
        pltpu.VMEM((1024,), jnp.int32),
        pltpu.VMEM((1024,), jnp.int32),
        pltpu.VMEM((_ROWS, 128), jnp.int32),
        pltpu.VMEM((_ROWS, 128), jnp.float32),
        pltpu.SemaphoreType.DMA,
    ],
)
def _robez_sc(idx_hbm, t0_hbm, t1_hbm, w_hbm, out_hbm,
              idx_v, t0_v, t1_v, gidx_v, rows_v, sem):
    wid = lax.axis_index("c") * 16 + lax.axis_index("s")
    pltpu.sync_copy(idx_hbm.at[pl.ds(wid * _BPW, _BPW)], idx_v)
    pltpu.sync_copy(t0_hbm, t0_v)
    pltpu.sync_copy(t1_hbm, t1_v)

    iota64 = lax.iota(jnp.int32, 16) * jnp.int32(_DIM)

    def chunk(c, carry):
        idxv = idx_v[pl.ds(c * 16, 16)]
        hi = idxv >> 10
        lo = idxv & 1023
        s = plsc.load_gather(t1_v, [hi]) + plsc.load_gather(t0_v, [lo])
        h = s + jnp.where(s >= _PBIAS, _R31, _MIN32)
        c8 = c * jnp.int32(8)
        for d in range(_DIM):
            u = h + jnp.int32(_KD[d])
            t = u + jnp.where(u >= _PBIAS, _R31, _MIN32)
            g = (t >> 20) + (t & jnp.int32(_M))
            g = jnp.where(g >= jnp.int32(_M), g - jnp.int32(_M), g)
            pre = iota64 + jnp.int32(d)
            plsc.store_scatter(gidx_v, [(pre >> 7) + c8, pre & 127], g)
        return carry
    lax.fori_loop(jnp.int32(0), jnp.int32(_NCH), chunk, jnp.int32(0))

    def dma_grp(r8, carry):
        r0 = r8 * 8
        cps = [
            pltpu.async_copy(w_hbm.at[gidx_v.at[r0 + j]], rows_v.at[r0 + j], sem)
            for j in range(8)
        ]
        for cp in cps:
            cp.wait()
        return carry
    lax.fori_loop(jnp.int32(0), jnp.int32(_ROWS // 8), dma_grp, jnp.int32(0))

    pltpu.sync_copy(rows_v, out_hbm.at[pl.ds(wid * _ROWS, _ROWS), :])


def kernel(indices, weight):
    idx32 = indices.astype(jnp.int32)
    out = _robez_sc(idx32, _T0, _T1B, weight)
    return out.reshape(_BATCH, _DIM)


# speedup vs baseline: 31.0989x; 1.2228x over previous
import functools

import numpy as np
import jax
import jax.numpy as jnp
from jax import lax
from jax.experimental import pallas as pl
from jax.experimental.pallas import tpu as pltpu
from jax.experimental.pallas import tpu_sc as plsc

_DIM = 64
_WSIZE = 1_048_576
_BATCH = 16_384
_M = _WSIZE - 1

_NW = 32
_BPW = _BATCH // _NW
_NCH = _BPW // 16
_ROWS = _BPW * _DIM // 128


def _hash_constants():
    r = np.random.RandomState(1024)
    rn = np.concatenate(
        [np.array([2038074743]), r.randint(0, 2038074743, (10,))]
    ).astype(np.int64)
    return int(rn[0]), int(rn[1]), int(rn[2]), int(rn[3])


_P, _A, _B, _C = _hash_constants()
_KD = [int((((d + 1) * _B + _C) % _P) - 2**31) for d in range(_DIM)]
_T0 = np.array([(k * _A) % _P for k in range(1024)], dtype=np.int64).astype(np.int32)
_T1B = np.array(
    [((k * 1024 * _A) % _P) - 2**31 for k in range(1024)], dtype=np.int64
).astype(np.int32)
_PBIAS = np.int32(_P - 2**31)
_R31 = np.int32(2**31 - _P)
_MIN32 = np.int32(-(2**31))

_mesh = plsc.VectorSubcoreMesh(core_axis_name="c", subcore_axis_name="s")


@functools.partial(
    pl.kernel,
    out_type=jax.ShapeDtypeStruct((_BATCH * _DIM // 128, 128), jnp.float32),
    mesh=_mesh,
    compiler_params=pltpu.CompilerParams(needs_layout_passes=False),
    scratch_types=[
        pltpu.VMEM((_BPW,), jnp.int32),
        pltpu.VMEM((1024,), jnp.int32),
        pltpu.VMEM((1024,), jnp.int32),
        pltpu.VMEM((_ROWS, 128), jnp.int32),
        pltpu.VMEM((_ROWS, 128), jnp.float32),
        pltpu.SemaphoreType.DMA,
    ],
)
def _robez_sc(idx_hbm, t0_hbm, t1_hbm, w_hbm, out_hbm,
              idx_v, t0_v, t1_v, gidx_v, rows_v, sem):
    wid = lax.axis_index("c") * 16 + lax.axis_index("s")
    pltpu.sync_copy(idx_hbm.at[pl.ds(wid * _BPW, _BPW)], idx_v)
    pltpu.sync_copy(t0_hbm, t0_v)
    pltpu.sync_copy(t1_hbm, t1_v)

    iota = lax.iota(jnp.int32, 16)
    rowbase = iota >> 1
    colbase = (iota & 1) * jnp.int32(_DIM)

    def chunk(c, carry):
        idxv = idx_v[pl.ds(c * 16, 16)]
        hi = idxv >> 10
        lo = idxv & 1023
        s = plsc.load_gather(t1_v, [hi]) + plsc.load_gather(t0_v, [lo])
        h = s + jnp.where(s >= _PBIAS, _R31, _MIN32)
        row = rowbase + c * jnp.int32(8)
        for d in range(_DIM):
            u = h + jnp.int32(_KD[d])
            t = u + jnp.where(u >= _PBIAS, _R31, _MIN32)
            g = (t >> 20) + (t & jnp.int32(_M))
            g = jnp.where(g >= jnp.int32(_M), g - jnp.int32(_M), g)
            plsc.store_scatter(gidx_v, [row, colbase + jnp.int32(d)], g)
        r0 = c * 8
        for j in range(8):
            pltpu.async_copy(w_hbm.at[gidx_v.at[r0 + j]], rows_v.at[r0 + j], sem)

        @pl.when(c > 0)
        def _drain_prev():
            for j in range(8):
                pltpu.make_async_copy(
                    w_hbm.at[gidx_v.at[r0 - 8 + j]], rows_v.at[r0 - 8 + j], sem
                ).wait()
        return carry
    lax.fori_loop(jnp.int32(0), jnp.int32(_NCH), chunk, jnp.int32(0))

    last = jnp.int32((_NCH - 1) * 8)
    for j in range(8):
        pltpu.make_async_copy(
            w_hbm.at[gidx_v.at[last + j]], rows_v.at[last + j], sem
        ).wait()

    pltpu.sync_copy(rows_v, out_hbm.at[pl.ds(wid * _ROWS, _ROWS), :])


def kernel(indices, weight):
    idx32 = indices.astype(jnp.int32)
    out = _robez_sc(idx32, _T0, _T1B, weight)
    return out.reshape(_BATCH, _DIM)


# speedup vs baseline: 34.9645x; 1.1243x over previous
import functools

import numpy as np
import jax
import jax.numpy as jnp
from jax import lax
from jax.experimental import pallas as pl
from jax.experimental.pallas import tpu as pltpu
from jax.experimental.pallas import tpu_sc as plsc

_DIM = 64
_WSIZE = 1_048_576
_BATCH = 16_384
_M = _WSIZE - 1

_NW = 32
_BPW = _BATCH // _NW
_NCH = _BPW // 16
_ROWS = _BPW * _DIM // 128


def _hash_constants():
    r = np.random.RandomState(1024)
    rn = np.concatenate(
        [np.array([2038074743]), r.randint(0, 2038074743, (10,))]
    ).astype(np.int64)
    return int(rn[0]), int(rn[1]), int(rn[2]), int(rn[3])


_P, _A, _B, _C = _hash_constants()
_KD = [int((((d + 1) * _B + _C) % _P) - 2**31) for d in range(_DIM)]
_T0 = np.array([(k * _A) % _P for k in range(1024)], dtype=np.int64).astype(np.int32)
_T1B = np.array(
    [((k * 1024 * _A) % _P) - 2**31 for k in range(1024)], dtype=np.int64
).astype(np.int32)
_PBIAS = np.int32(_P - 2**31)
_R31 = np.int32(2**31 - _P)
_MIN32 = np.int32(-(2**31))

_mesh = plsc.VectorSubcoreMesh(core_axis_name="c", subcore_axis_name="s")


@functools.partial(
    pl.kernel,
    out_type=jax.ShapeDtypeStruct((_BATCH * _DIM // 128, 128), jnp.float32),
    mesh=_mesh,
    compiler_params=pltpu.CompilerParams(needs_layout_passes=False),
    scratch_types=[
        pltpu.VMEM((_BPW,), jnp.int32),
        pltpu.VMEM((1024,), jnp.int32),
        pltpu.VMEM((1024,), jnp.int32),
        pltpu.VMEM((_ROWS, 128), jnp.int32),
        pltpu.VMEM((_ROWS, 128), jnp.float32),
        pltpu.SemaphoreType.DMA,
    ],
)
def _robez_sc(idx_hbm, t0_hbm, t1_hbm, w_hbm, out_hbm,
              idx_v, t0_v, t1_v, gidx_v, rows_v, sem):
    wid = lax.axis_index("c") * 16 + lax.axis_index("s")
    pltpu.sync_copy(idx_hbm.at[pl.ds(wid * _BPW, _BPW)], idx_v)
    pltpu.sync_copy(t0_hbm, t0_v)
    pltpu.sync_copy(t1_hbm, t1_v)

    iota = lax.iota(jnp.int32, 16)
    rowbase = iota >> 1
    colbase = (iota & 1) * jnp.int32(_DIM)

    def chunk(c, carry):
        idxv = idx_v[pl.ds(c * 16, 16)]
        hi = idxv >> 10
        lo = idxv & 1023
        s = plsc.load_gather(t1_v, [hi]) + plsc.load_gather(t0_v, [lo])
        h = s + jnp.where(s >= _PBIAS, _R31, _MIN32)
        row = rowbase + c * jnp.int32(8)
        for d in range(_DIM):
            u = h + jnp.int32(_KD[d])
            t = u + jnp.where(u >= _PBIAS, _R31, _MIN32)
            g = (t >> 20) + (t & jnp.int32(_M))
            g = jnp.where(g >= jnp.int32(_M), g - jnp.int32(_M), g)
            plsc.store_scatter(gidx_v, [row, colbase + jnp.int32(d)], g)
        r0 = c * 8
        for j in range(8):
            pltpu.async_copy(w_hbm.at[gidx_v.at[r0 + j]], rows_v.at[r0 + j], sem)

        @pl.when(c > 2)
        def _drain_prev():
            for j in range(8):
                pltpu.make_async_copy(
                    w_hbm.at[gidx_v.at[r0 - 24 + j]], rows_v.at[r0 - 24 + j], sem
                ).wait()
        return carry
    lax.fori_loop(jnp.int32(0), jnp.int32(_NCH), chunk, jnp.int32(0))

    for r in range(8 * (_NCH - 3), 8 * _NCH):
        pltpu.make_async_copy(
            w_hbm.at[gidx_v.at[jnp.int32(r)]], rows_v.at[jnp.int32(r)], sem
        ).wait()

    pltpu.sync_copy(rows_v, out_hbm.at[pl.ds(wid * _ROWS, _ROWS), :])


def kernel(indices, weight):
    idx32 = indices.astype(jnp.int32)
    out = _robez_sc(idx32, _T0, _T1B, weight)
    return out.reshape(_BATCH, _DIM)


# speedup vs baseline: 38.5639x; 1.1029x over previous
import functools

import numpy as np
import jax
import jax.numpy as jnp
from jax import lax
from jax.experimental import pallas as pl
from jax.experimental.pallas import tpu as pltpu
from jax.experimental.pallas import tpu_sc as plsc

_DIM = 64
_WSIZE = 1_048_576
_BATCH = 16_384
_M = _WSIZE - 1

_NW = 32
_BPW = _BATCH // _NW
_NCH = _BPW // 16
_ROWS = _BPW * _DIM // 128


def _hash_constants():
    r = np.random.RandomState(1024)
    rn = np.concatenate(
        [np.array([2038074743]), r.randint(0, 2038074743, (10,))]
    ).astype(np.int64)
    return int(rn[0]), int(rn[1]), int(rn[2]), int(rn[3])


_P, _A, _B, _C = _hash_constants()
_KD = [int((((d + 1) * _B + _C) % _P) - 2**31) for d in range(_DIM)]
_T0 = np.array([(k * _A) % _P for k in range(1024)], dtype=np.int64).astype(np.int32)
_T1B = np.array(
    [((k * 1024 * _A) % _P) - 2**31 for k in range(1024)], dtype=np.int64
).astype(np.int32)
_PBIAS = np.int32(_P - 2**31)
_R31 = np.int32(2**31 - _P)
_MIN32 = np.int32(-(2**31))

_mesh = plsc.VectorSubcoreMesh(core_axis_name="c", subcore_axis_name="s")


@functools.partial(
    pl.kernel,
    out_type=jax.ShapeDtypeStruct((_BATCH * _DIM // 128, 128), jnp.float32),
    mesh=_mesh,
    compiler_params=pltpu.CompilerParams(needs_layout_passes=False),
    scratch_types=[
        pltpu.VMEM((_BPW,), jnp.int32),
        pltpu.VMEM((1024,), jnp.int32),
        pltpu.VMEM((1024,), jnp.int32),
        pltpu.VMEM((_ROWS, 128), jnp.int32),
        pltpu.VMEM((_ROWS, 128), jnp.float32),
        pltpu.SemaphoreType.DMA,
    ],
)
def _robez_sc(idx_hbm, t0_hbm, t1_hbm, w_hbm, out_hbm,
              idx_v, t0_v, t1_v, gidx_v, rows_v, sem):
    wid = lax.axis_index("c") * 16 + lax.axis_index("s")
    pltpu.sync_copy(idx_hbm.at[pl.ds(wid * _BPW, _BPW)], idx_v)
    pltpu.sync_copy(t0_hbm, t0_v)
    pltpu.sync_copy(t1_hbm, t1_v)

    iota = lax.iota(jnp.int32, 16)
    rowbase = iota >> 1
    colbase = (iota & 1) * jnp.int32(_DIM)

    def chunk(c, carry):
        idxv = idx_v[pl.ds(c * 16, 16)]
        hi = idxv >> 10
        lo = idxv & 1023
        s = plsc.load_gather(t1_v, [hi]) + plsc.load_gather(t0_v, [lo])
        h = s + jnp.where(s >= _PBIAS, _R31, _MIN32)
        row = rowbase + c * jnp.int32(8)
        for d in range(_DIM):
            u = h + jnp.int32(_KD[d])
            t = u + jnp.where(u >= _PBIAS, _R31, _MIN32)
            g = (t >> 20) + (t & jnp.int32(_M))
            g = jnp.where(g >= jnp.int32(_M), g - jnp.int32(_M), g)
            plsc.store_scatter(gidx_v, [row, colbase + jnp.int32(d)], g)
        r0 = c * 8
        for j in range(8):
            pltpu.async_copy(w_hbm.at[gidx_v.at[r0 + j]], rows_v.at[r0 + j], sem)

        return carry
    lax.fori_loop(jnp.int32(0), jnp.int32(_NCH), chunk, jnp.int32(0))

    def drain(r, carry):
        pltpu.make_async_copy(w_hbm.at[gidx_v.at[r]], rows_v.at[r], sem).wait()
        return carry
    lax.fori_loop(jnp.int32(0), jnp.int32(_ROWS), drain, jnp.int32(0))

    pltpu.sync_copy(rows_v, out_hbm.at[pl.ds(wid * _ROWS, _ROWS), :])


def kernel(indices, weight):
    idx32 = indices.astype(jnp.int32)
    out = _robez_sc(idx32, _T0, _T1B, weight)
    return out.reshape(_BATCH, _DIM)
